# Initial kernel scaffold; baseline (speedup 1.0000x reference)
import jax
import jax.numpy as jnp
from jax.experimental import pallas as pl


def kernel(embeddings, logits, targets, alt_logits, indices, centroids, u, target_probs):
    raise NotImplementedError("write your pallas kernel here")


# speedup vs baseline: 1.0588x; 1.0588x over previous
import functools

import jax
import jax.numpy as jnp
from jax import lax
from jax.experimental import pallas as pl
from jax.experimental.pallas import tpu as pltpu
from jax.experimental.pallas import tpu_sc as plsc

NUM_CLASSES = 1000
EMB_DIM = 128
NUM_SAMPLES = 100000
BATCH = 4096
LAMBDA_C = 1.0
LAMBDA_B = 1.0
LAMBDA_ELR = 3.0
EMA = 0.9

ROWS = 256
GRID = BATCH // ROWS


@functools.cache
def _make_sc_gather():
    info = plsc.get_sparse_core_info()
    ncores, nsub = info.num_cores, info.num_subcores
    nworkers = ncores * nsub
    bpw = BATCH // nworkers
    mesh = plsc.VectorSubcoreMesh(core_axis_name="c", subcore_axis_name="s")

    @functools.partial(
        pl.kernel,
        mesh=mesh,
        out_type=jax.ShapeDtypeStruct((BATCH, NUM_CLASSES), jnp.float32),
        compiler_params=pltpu.CompilerParams(use_tc_tiling_on_sc=False),
        scratch_types=[
            pltpu.VMEM((bpw,), jnp.int32),
            pltpu.VMEM((bpw, NUM_CLASSES), jnp.float32),
            pltpu.SemaphoreType.DMA,
        ],
    )
    def gather_k(table_hbm, idx_hbm, out_hbm, idx_v, rows_v, sem):
        wid = lax.axis_index("s") * ncores + lax.axis_index("c")
        base = wid * bpw
        pltpu.sync_copy(idx_hbm.at[pl.ds(base, bpw)], idx_v)
        pltpu.async_copy(table_hbm.at[idx_v], rows_v, sem).wait()
        pltpu.sync_copy(rows_v, out_hbm.at[pl.ds(base, bpw)])

    return gather_k


def _loss_body(logit_ref, alt_ref, q_ref, t_ref, emb_ref, cent_ref, u_ref,
               out_ref, acc):
    i = pl.program_id(0)

    x = logit_ref[...]
    a = alt_ref[...]

    m1 = jnp.max(x, axis=1, keepdims=True)
    e1 = jnp.exp(x - m1)
    s1 = jnp.sum(e1, axis=1, keepdims=True)
    p1 = e1 / s1
    logp1 = (x - m1) - jnp.log(s1)

    m2 = jnp.max(a, axis=1, keepdims=True)
    e2 = jnp.exp(a - m2)
    s2 = jnp.sum(e2, axis=1, keepdims=True)
    p2 = e2 / s2
    logp2 = (a - m2) - jnp.log(s2)

    t = t_ref[...]
    cls = lax.broadcasted_iota(jnp.int32, (ROWS, NUM_CLASSES), 1)
    oh = (cls == t).astype(jnp.float32)

    ce_part = jnp.sum(logp1 * oh)

    embv = emb_ref[...]
    cent = cent_ref[...]
    dot = lax.dot_general(embv, cent, (((1,), (1,)), ((), ())),
                          preferred_element_type=jnp.float32)
    en = jnp.sqrt(jnp.sum(embv * embv, axis=1, keepdims=True))
    ones = jnp.ones((1, EMB_DIM), jnp.float32)
    cn2 = lax.dot_general(ones, cent * cent, (((1,), (1,)), ((), ())),
                          preferred_element_type=jnp.float32)
    denom = jnp.maximum(en * jnp.sqrt(cn2), 1e-8)
    sim = dot / denom
    ms = jnp.max(sim, axis=1, keepdims=True)
    es = jnp.exp(sim - ms)
    soft = es / jnp.sum(es, axis=1, keepdims=True)

    s2t_part = jnp.sum(soft * oh)
    softsq_part = jnp.sum(soft * soft)

    klc_part = jnp.sum((p1 - p2) * (logp1 - logp2))
    logavg_part = jnp.sum(jnp.log(0.5 * (p1 + p2)))

    q = EMA * q_ref[...] + (1.0 - EMA) * p1
    elr_part = jnp.sum(p1 * jnp.log(1.0 - q + 1e-6))

    @pl.when(i == 0)
    def _init():
        acc[0] = ce_part
        acc[1] = s2t_part
        acc[2] = softsq_part
        acc[3] = klc_part
        acc[4] = logavg_part
        acc[5] = elr_part

    @pl.when(i > 0)
    def _accum():
        acc[0] += ce_part
        acc[1] += s2t_part
        acc[2] += softsq_part
        acc[3] += klc_part
        acc[4] += logavg_part
        acc[5] += elr_part

    @pl.when(i == GRID - 1)
    def _finish():
        bf = float(BATCH)
        cf = float(NUM_CLASSES)
        ce = -acc[0] / bf
        mse = (acc[2] - 2.0 * acc[1] + bf) / (bf * cf)
        reg = (1.0 - acc[1] / bf) * u_ref[0, 0]
        klc = acc[3] / bf
        klb = -jnp.log(cf) - acc[4] / (bf * cf)
        elr = -acc[5] / bf
        out_ref[0, 0] = (ce + mse + reg + LAMBDA_C * klc + LAMBDA_B * klb
                         + LAMBDA_ELR * elr)


def _loss_tc(logits, alt_logits, qrows, targets2d, embeddings, centroids, u2d):
    return pl.pallas_call(
        _loss_body,
        grid=(GRID,),
        in_specs=[
            pl.BlockSpec((ROWS, NUM_CLASSES), lambda i: (i, 0)),
            pl.BlockSpec((ROWS, NUM_CLASSES), lambda i: (i, 0)),
            pl.BlockSpec((ROWS, NUM_CLASSES), lambda i: (i, 0)),
            pl.BlockSpec((ROWS, 1), lambda i: (i, 0)),
            pl.BlockSpec((ROWS, EMB_DIM), lambda i: (i, 0)),
            pl.BlockSpec((NUM_CLASSES, EMB_DIM), lambda i: (0, 0)),
            pl.BlockSpec(memory_space=pltpu.SMEM),
        ],
        out_specs=pl.BlockSpec(memory_space=pltpu.SMEM),
        out_shape=jax.ShapeDtypeStruct((1, 1), jnp.float32),
        scratch_shapes=[pltpu.SMEM((8,), jnp.float32)],
    )(logits, alt_logits, qrows, targets2d, embeddings, centroids, u2d)


def kernel(embeddings, logits, targets, alt_logits, indices, centroids, u,
           target_probs):
    qrows = _make_sc_gather()(target_probs, indices.astype(jnp.int32))
    loss = _loss_tc(
        logits,
        alt_logits,
        qrows,
        targets.astype(jnp.int32).reshape(BATCH, 1),
        embeddings,
        centroids,
        u.reshape(1, 1).astype(jnp.float32),
    )
    return loss.reshape(1)
